# baseline (speedup 1.0000x reference)
import functools

import jax
import jax.numpy as jnp
from jax import lax
from jax.experimental import pallas as pl
from jax.experimental.pallas import tpu as pltpu
from jax.experimental.pallas import tpu_sc as plsc

_NUM_CLASSES = 1000000
_OUT_DIM = 64
_BATCH = 16384
_L = 16
_NC = 2
_NS = 16
_NW = _NC * _NS
_B_PER_W = _BATCH // _NW
_NG = _B_PER_W // _L

_PAIR_BLOCK = 512
_N_PAIR = 500224
_TC_GRID = _N_PAIR // _PAIR_BLOCK

_mesh = plsc.VectorSubcoreMesh(core_axis_name="c", subcore_axis_name="s")


def _pack_body(lo_ref, hi_ref, p_ref):
    p_ref[:, 0:_OUT_DIM] = lo_ref[...].T
    p_ref[:, _OUT_DIM:2 * _OUT_DIM] = hi_ref[...].T


def _pack_pairs(table_t):
    return pl.pallas_call(
        _pack_body,
        grid=(_TC_GRID,),
        in_specs=[
            pl.BlockSpec((_OUT_DIM, _PAIR_BLOCK), lambda g: (0, g)),
            pl.BlockSpec((_OUT_DIM, _PAIR_BLOCK),
                         lambda g: (0, g + _TC_GRID)),
        ],
        out_specs=pl.BlockSpec((_PAIR_BLOCK, 2 * _OUT_DIM), lambda g: (g, 0)),
        out_shape=jax.ShapeDtypeStruct((_N_PAIR, 2 * _OUT_DIM), jnp.float32),
    )(table_t, table_t)


@functools.partial(
    pl.kernel,
    mesh=_mesh,
    out_type=jax.ShapeDtypeStruct((_BATCH, _OUT_DIM), jnp.float32),
    scratch_types=[
        pltpu.VMEM((_B_PER_W,), jnp.int32),
        pltpu.VMEM((_B_PER_W,), jnp.int32),
        pltpu.VMEM((_B_PER_W,), jnp.int32),
        pltpu.VMEM((_L,), jnp.int32),
        pltpu.VMEM((_B_PER_W // 2, 2 * _OUT_DIM), jnp.float32),
        pltpu.VMEM((_B_PER_W, _OUT_DIM), jnp.float32),
        pltpu.SemaphoreType.DMA,
        pltpu.SemaphoreType.DMA,
    ],
    compiler_params=pltpu.CompilerParams(needs_layout_passes=False),
)
def _embed(labels_hbm, train_hbm, drop_hbm, pairs_hbm, out_hbm,
           idx_v, drop_v, off_v, train_v, prows_v, rows_v, sem, sem2):
    wid = lax.axis_index("s") * _NC + lax.axis_index("c")
    base = wid * _B_PER_W
    pltpu.sync_copy(labels_hbm.at[pl.ds(base, _B_PER_W)], idx_v)
    pltpu.sync_copy(drop_hbm.at[pl.ds(base, _B_PER_W)], drop_v)
    pltpu.sync_copy(train_hbm, train_v)
    trn = train_v[...]
    null_row = jnp.full((_L,), _NUM_CLASSES, dtype=jnp.int32)
    for i in range(_NG):
        sl = pl.ds(i * _L, _L)
        adj = jnp.where((trn != 0) & (drop_v[sl] != 0), null_row, idx_v[sl])
        hi = (adj >= _N_PAIR).astype(jnp.int32)
        idx_v[sl] = adj - hi * _N_PAIR
        off_v[sl] = hi * _OUT_DIM

    lanes = lax.iota(jnp.int32, _L)
    half = _B_PER_W // 2
    for h in range(2):
        pltpu.async_copy(
            pairs_hbm.at[idx_v.at[pl.ds(h * half, half)]], prows_v, sem
        ).wait()

        @pl.loop(0, half // _L)
        def _(t):
            rows16 = lanes + t * _L
            off16 = off_v[pl.ds(h * half + t * _L, _L)]
            out16 = rows16 + h * half
            for c in range(_OUT_DIM):
                vals = plsc.load_gather(prows_v, [rows16, off16 + c])
                plsc.store_scatter(
                    rows_v, [out16, jnp.full((_L,), c, jnp.int32)], vals)

    pltpu.async_copy(rows_v, out_hbm.at[pl.ds(base, _B_PER_W)], sem2).wait()


def kernel(labels, train, force_drop_ids, table):
    labels32 = labels.astype(jnp.int32)
    drop32 = force_drop_ids.astype(jnp.int32)
    train_vec = jnp.full((_L,), jnp.asarray(train, dtype=jnp.int32))
    pairs = _pack_pairs(table.T)
    return _embed(labels32, train_vec, drop32, pairs)


# speedup vs baseline: 1.0035x; 1.0035x over previous
import functools

import jax
import jax.numpy as jnp
from jax import lax
from jax.experimental import pallas as pl
from jax.experimental.pallas import tpu as pltpu
from jax.experimental.pallas import tpu_sc as plsc

_NUM_CLASSES = 1000000
_OUT_DIM = 64
_BATCH = 16384
_L = 16
_NC = 2
_NS = 16
_NW = _NC * _NS
_B_PER_W = _BATCH // _NW
_NG = _B_PER_W // _L
_QROWS = 128

_PAIR_BLOCK = 512
_N_PAIR = 500224
_TC_GRID = _N_PAIR // _PAIR_BLOCK

_mesh = plsc.VectorSubcoreMesh(core_axis_name="c", subcore_axis_name="s")


def _pack_body(lo_ref, hi_ref, p_ref):
    p_ref[...] = jnp.concatenate([lo_ref[...].T, hi_ref[...].T], axis=1)


def _pack_pairs(table_t):
    return pl.pallas_call(
        _pack_body,
        grid=(_TC_GRID,),
        in_specs=[
            pl.BlockSpec((_OUT_DIM, _PAIR_BLOCK), lambda g: (0, g)),
            pl.BlockSpec((_OUT_DIM, _PAIR_BLOCK),
                         lambda g: (0, g + _TC_GRID)),
        ],
        out_specs=pl.BlockSpec((_PAIR_BLOCK, 2 * _OUT_DIM), lambda g: (g, 0)),
        out_shape=jax.ShapeDtypeStruct((_N_PAIR, 2 * _OUT_DIM), jnp.float32),
    )(table_t, table_t)


@functools.partial(
    pl.kernel,
    mesh=_mesh,
    out_type=jax.ShapeDtypeStruct((_BATCH, _OUT_DIM), jnp.float32),
    scratch_types=[
        pltpu.VMEM((_B_PER_W,), jnp.int32),
        pltpu.VMEM((_B_PER_W,), jnp.int32),
        pltpu.VMEM((_B_PER_W,), jnp.int32),
        pltpu.VMEM((_L,), jnp.int32),
        pltpu.VMEM((_QROWS, 2 * _OUT_DIM), jnp.float32),
        pltpu.VMEM((_QROWS, 2 * _OUT_DIM), jnp.float32),
        pltpu.VMEM((_QROWS, 2 * _OUT_DIM), jnp.float32),
        pltpu.VMEM((_QROWS, _OUT_DIM), jnp.float32),
        pltpu.SemaphoreType.DMA,
        pltpu.SemaphoreType.DMA,
        pltpu.SemaphoreType.DMA,
        pltpu.SemaphoreType.DMA,
    ],
    compiler_params=pltpu.CompilerParams(needs_layout_passes=False),
)
def _embed(labels_hbm, train_hbm, drop_hbm, pairs_hbm, out_hbm,
           idx_v, drop_v, off_v, train_v, pb0, pb1, pb2, rows_v,
           qs0, qs1, qs2, sem2):
    wid = lax.axis_index("s") * _NC + lax.axis_index("c")
    base = wid * _B_PER_W
    pltpu.sync_copy(labels_hbm.at[pl.ds(base, _B_PER_W)], idx_v)
    pltpu.sync_copy(drop_hbm.at[pl.ds(base, _B_PER_W)], drop_v)
    pltpu.sync_copy(train_hbm, train_v)
    trn = train_v[...]
    null_row = jnp.full((_L,), _NUM_CLASSES, dtype=jnp.int32)
    for i in range(_NG):
        sl = pl.ds(i * _L, _L)
        adj = jnp.where((trn != 0) & (drop_v[sl] != 0), null_row, idx_v[sl])
        hi = (adj >= _N_PAIR).astype(jnp.int32)
        idx_v[sl] = adj - hi * _N_PAIR
        off_v[sl] = hi * _OUT_DIM

    lanes = lax.iota(jnp.int32, _L)
    bufs = (pb0, pb1, pb2)
    sems = (qs0, qs1, qs2)
    nq = _B_PER_W // _QROWS

    def fire(q):
        pltpu.async_copy(
            pairs_hbm.at[idx_v.at[pl.ds(q * _QROWS, _QROWS)]],
            bufs[q % 3], sems[q % 3])

    for q in range(min(3, nq)):
        fire(q)
    for q in range(nq):
        pltpu.make_async_copy(
            pairs_hbm.at[idx_v.at[pl.ds(q * _QROWS, _QROWS)]],
            bufs[q % 3], sems[q % 3]).wait()
        buf = bufs[q % 3]

        @pl.loop(0, _QROWS // _L)
        def _(t):
            rows16 = lanes + t * _L
            off16 = off_v[pl.ds(q * _QROWS + t * _L, _L)]
            for c in range(_OUT_DIM):
                vals = plsc.load_gather(buf, [rows16, off16 + c])
                plsc.store_scatter(
                    rows_v, [rows16, jnp.full((_L,), c, jnp.int32)], vals)

        if q + 3 < nq:
            fire(q + 3)
        pltpu.async_copy(
            rows_v, out_hbm.at[pl.ds(base + q * _QROWS, _QROWS)], sem2
        ).wait()


def kernel(labels, train, force_drop_ids, table):
    labels32 = labels.astype(jnp.int32)
    drop32 = force_drop_ids.astype(jnp.int32)
    train_vec = jnp.full((_L,), jnp.asarray(train, dtype=jnp.int32))
    pairs = _pack_pairs(table.T)
    return _embed(labels32, train_vec, drop32, pairs)


# speedup vs baseline: 1.2070x; 1.2028x over previous
import functools

import jax
import jax.numpy as jnp
from jax import lax
from jax.experimental import pallas as pl
from jax.experimental.pallas import tpu as pltpu
from jax.experimental.pallas import tpu_sc as plsc

_NUM_CLASSES = 1000000
_ROWS = _NUM_CLASSES + 1
_OUT_DIM = 64
_BATCH = 16384
_L = 16
_NC = 2
_NS = 16
_NW = _NC * _NS
_B_PER_W = _BATCH // _NW
_NG = _B_PER_W // _L
_NCHUNK = 4
_CROWS = 250048

_mesh = plsc.VectorSubcoreMesh(core_axis_name="c", subcore_axis_name="s")


def _make_embed(cbase, csize):
    @functools.partial(
        pl.kernel,
        mesh=_mesh,
        out_type=jax.ShapeDtypeStruct((_BATCH, _OUT_DIM), jnp.float32),
        scratch_types=[
            pltpu.VMEM((_B_PER_W,), jnp.int32),
            pltpu.VMEM((_B_PER_W,), jnp.int32),
            pltpu.VMEM((_B_PER_W,), jnp.int32),
            pltpu.VMEM((_L,), jnp.int32),
            pltpu.VMEM((_B_PER_W, _OUT_DIM), jnp.float32),
            pltpu.SemaphoreType.DMA,
            pltpu.SemaphoreType.DMA,
        ],
        compiler_params=pltpu.CompilerParams(needs_layout_passes=False),
    )
    def _embed(labels_hbm, train_hbm, drop_hbm, chunk_hbm, out_hbm,
               idx_v, drop_v, val_v, train_v, rows_v, sem, sem2):
        wid = lax.axis_index("s") * _NC + lax.axis_index("c")
        base = wid * _B_PER_W
        pltpu.sync_copy(labels_hbm.at[pl.ds(base, _B_PER_W)], idx_v)
        pltpu.sync_copy(drop_hbm.at[pl.ds(base, _B_PER_W)], drop_v)
        pltpu.sync_copy(train_hbm, train_v)
        trn = train_v[...]
        null_row = jnp.full((_L,), _NUM_CLASSES, dtype=jnp.int32)
        zeros16 = jnp.zeros((_L,), jnp.float32)
        cnt = jnp.int32(0)
        for i in range(_NG):
            sl = pl.ds(i * _L, _L)
            adj = jnp.where((trn != 0) & (drop_v[sl] != 0),
                            null_row, idx_v[sl])
            local = adj - cbase
            valid = (local >= 0) & (local < csize)
            idx_v[sl] = jnp.where(valid, local, 0)
            val_v[sl] = jnp.where(valid, jnp.full((_L,), 1, jnp.int32),
                                  jnp.full((_L,), 0, jnp.int32))
            cnt = cnt + plsc.all_reduce_population_count(valid)[0]

        @pl.loop(0, _B_PER_W)
        def _(j):
            for m in range(_OUT_DIM // _L):
                rows_v[j, pl.ds(m * _L, _L)] = zeros16

        @pl.loop(0, _NG)
        def _(g):
            local = idx_v[pl.ds(g * _L, _L)]
            valid = val_v[pl.ds(g * _L, _L)]
            for k in range(_L):
                @pl.when(valid[k] != 0)
                def _():
                    pltpu.async_copy(
                        chunk_hbm.at[pl.ds(local[k], 1), :],
                        rows_v.at[pl.ds(g * _L + k, 1), :], sem)

        @pl.loop(0, _B_PER_W)
        def _(j):
            @pl.when(j < cnt)
            def _():
                pltpu.make_async_copy(
                    chunk_hbm.at[pl.ds(0, 1), :],
                    rows_v.at[pl.ds(0, 1), :], sem).wait()

        pltpu.async_copy(rows_v, out_hbm.at[pl.ds(base, _B_PER_W)],
                         sem2).wait()

    return _embed


def kernel(labels, train, force_drop_ids, table):
    labels32 = labels.astype(jnp.int32)
    drop32 = force_drop_ids.astype(jnp.int32)
    train_vec = jnp.full((_L,), jnp.asarray(train, dtype=jnp.int32))
    out = None
    for c in range(_NCHUNK):
        cbase = c * _CROWS
        csize = min(_CROWS, _ROWS - cbase)
        chunk = lax.slice(table, (cbase, 0), (cbase + csize, _OUT_DIM))
        part = _make_embed(cbase, csize)(labels32, train_vec, drop32, chunk)
        out = part if out is None else out + part
    return out


# speedup vs baseline: 1.8785x; 1.5563x over previous
import functools

import jax
import jax.numpy as jnp
from jax import lax
from jax.experimental import pallas as pl
from jax.experimental.pallas import tpu as pltpu
from jax.experimental.pallas import tpu_sc as plsc

_NUM_CLASSES = 1000000
_ROWS = _NUM_CLASSES + 1
_OUT_DIM = 64
_BATCH = 16384
_L = 16
_NC = 2
_NS = 16
_NW = _NC * _NS
_B_PER_W = _BATCH // _NW
_NG = _B_PER_W // _L
_NCHUNK = 4
_CROWS = 250048

_mesh = plsc.VectorSubcoreMesh(core_axis_name="c", subcore_axis_name="s")


def _make_embed(cbase, csize, has_null):
    @functools.partial(
        pl.kernel,
        mesh=_mesh,
        out_type=jax.ShapeDtypeStruct((_BATCH, _OUT_DIM), jnp.float32),
        scratch_types=[
            pltpu.VMEM((_B_PER_W,), jnp.int32),
            pltpu.VMEM((_B_PER_W,), jnp.int32),
            pltpu.VMEM((_B_PER_W,), jnp.int32),
            pltpu.VMEM((_B_PER_W,), jnp.int32),
            pltpu.VMEM((_L,), jnp.int32),
            pltpu.VMEM((1, _OUT_DIM), jnp.float32),
            pltpu.VMEM((_B_PER_W, _OUT_DIM), jnp.float32),
            pltpu.SemaphoreType.DMA,
            pltpu.SemaphoreType.DMA,
        ],
        compiler_params=pltpu.CompilerParams(needs_layout_passes=False),
    )
    def _embed(labels_hbm, train_hbm, drop_hbm, chunk_hbm, out_hbm,
               idx_v, drop_v, val_v, dr_v, train_v, null_v, rows_v,
               sem, sem2):
        wid = lax.axis_index("s") * _NC + lax.axis_index("c")
        base = wid * _B_PER_W
        pltpu.sync_copy(labels_hbm.at[pl.ds(base, _B_PER_W)], idx_v)
        pltpu.sync_copy(drop_hbm.at[pl.ds(base, _B_PER_W)], drop_v)
        pltpu.sync_copy(train_hbm, train_v)
        if has_null:
            pltpu.sync_copy(chunk_hbm.at[pl.ds(_NUM_CLASSES - cbase, 1), :],
                            null_v)
        trn = train_v[...]
        one16 = jnp.full((_L,), 1, jnp.int32)
        zero16 = jnp.full((_L,), 0, jnp.int32)
        zerof16 = jnp.zeros((_L,), jnp.float32)
        cnt = jnp.int32(0)
        for i in range(_NG):
            sl = pl.ds(i * _L, _L)
            dropped = (trn != 0) & (drop_v[sl] != 0)
            local = idx_v[sl] - cbase
            valid = (~dropped) & (local >= 0) & (local < csize)
            idx_v[sl] = jnp.where(valid, local, 0)
            val_v[sl] = jnp.where(valid, one16, zero16)
            dr_v[sl] = jnp.where(dropped, one16, zero16)
            cnt = cnt + plsc.all_reduce_population_count(valid)[0]

        @pl.loop(0, _B_PER_W)
        def _(j):
            for m in range(_OUT_DIM // _L):
                rows_v[j, pl.ds(m * _L, _L)] = zerof16

        @pl.loop(0, _NG)
        def _(g):
            local = idx_v[pl.ds(g * _L, _L)]
            valid = val_v[pl.ds(g * _L, _L)]
            for k in range(_L):
                @pl.when(valid[k] != 0)
                def _():
                    pltpu.async_copy(
                        chunk_hbm.at[pl.ds(local[k], 1), :],
                        rows_v.at[pl.ds(g * _L + k, 1), :], sem)

        if has_null:
            nulls = [null_v[0, pl.ds(m * _L, _L)]
                     for m in range(_OUT_DIM // _L)]

            @pl.loop(0, _NG)
            def _(g):
                dr = dr_v[pl.ds(g * _L, _L)]
                for k in range(_L):
                    @pl.when(dr[k] != 0)
                    def _():
                        for m in range(_OUT_DIM // _L):
                            rows_v[g * _L + k, pl.ds(m * _L, _L)] = nulls[m]

        @pl.loop(0, _B_PER_W)
        def _(j):
            @pl.when(j < cnt)
            def _():
                pltpu.make_async_copy(
                    chunk_hbm.at[pl.ds(0, 1), :],
                    rows_v.at[pl.ds(0, 1), :], sem).wait()

        pltpu.async_copy(rows_v, out_hbm.at[pl.ds(base, _B_PER_W)],
                         sem2).wait()

    return _embed


def kernel(labels, train, force_drop_ids, table):
    labels32 = labels.astype(jnp.int32)
    drop32 = force_drop_ids.astype(jnp.int32)
    train_vec = jnp.full((_L,), jnp.asarray(train, dtype=jnp.int32))
    out = None
    for c in range(_NCHUNK):
        cbase = c * _CROWS
        csize = min(_CROWS, _ROWS - cbase)
        has_null = cbase <= _NUM_CLASSES < cbase + csize
        chunk = lax.slice(table, (cbase, 0), (cbase + csize, _OUT_DIM))
        part = _make_embed(cbase, csize, has_null)(
            labels32, train_vec, drop32, chunk)
        out = part if out is None else out + part
    return out
